# Initial kernel scaffold; baseline (speedup 1.0000x reference)
import jax
import jax.numpy as jnp
from jax.experimental import pallas as pl


def kernel(x, tw, demand, edge_index, edge_attr, Wn, bn_b, g1, b1, We, be_b, g2, b2, Wfc, bfc, Wattn, battn):
    raise NotImplementedError("write your pallas kernel here")


# speedup vs baseline: 2.7424x; 2.7424x over previous
import functools

import jax
import jax.numpy as jnp
from jax import lax
from jax.experimental import pallas as pl
from jax.experimental.pallas import tpu as pltpu
from jax.experimental.pallas import tpu_sc as plsc

N = 10000
E = 320000
EMBED = 128
HE = 16
LAYERS = 3
R = 10112
K = 128
NCH = 157
EPT = NCH * K
E_PAD = 16 * EPT
ROWS_PT = R // 16
EPS_BN = 1e-5
EPS_DEN = 1e-16


def _prep_body(z_ref, attr_ref, p_ref, wnt_ref, wea_ref, battn_ref,
               h_ref, uw_ref):
    z = z_ref[...]
    y = jnp.dot(z, wnt_ref[...], preferred_element_type=jnp.float32)
    y = y + p_ref[3, :]
    m = jnp.mean(y, axis=0)
    v = jnp.mean(y * y, axis=0) - m * m
    h0 = (y - m) / jnp.sqrt(v + EPS_BN) * p_ref[4, :] + p_ref[5, :]
    h_ref[...] = jnp.concatenate(
        [h0, jnp.zeros((R - N, EMBED), jnp.float32)], axis=0)

    a2 = attr_ref[...]
    mean_a = jnp.sum(a2) / E
    var_a = jnp.sum(a2 * a2) / E - mean_a * mean_a
    we = p_ref[0, :HE]
    g2 = p_ref[1, :HE]
    b2 = p_ref[2, :HE]
    s16 = g2 * we / jnp.sqrt(var_a * we * we + EPS_BN)
    t16 = b2 - s16 * mean_a
    u_flat = jnp.dot(wea_ref[...], s16[:, None],
                     preferred_element_type=jnp.float32)[:, 0]
    w_flat = jnp.dot(wea_ref[...], t16[:, None],
                     preferred_element_type=jnp.float32)[:, 0] + battn_ref[0, :]
    uw_ref[...] = jnp.concatenate(
        [u_flat[None, :], w_flat[None, :],
         jnp.zeros((6, LAYERS * EMBED), jnp.float32)], axis=0)


_prep_call = pl.pallas_call(
    _prep_body,
    out_shape=[jax.ShapeDtypeStruct((R, EMBED), jnp.float32),
               jax.ShapeDtypeStruct((8, LAYERS * EMBED), jnp.float32)],
)


def _dense_body(with_update, *refs):
    if with_update:
        (h_ref, a0_ref, a1_ref, wfc_ref, wi_ref, wj_ref,
         bfc_ref, wv_ref, ho_ref, ts_ref, td_ref) = refs
        a0 = a0_ref[...]
        a1 = a1_ref[...]
        H = EMBED // 2
        upd = jnp.concatenate(
            [a0[:, :H] / (a0[:, H:] + EPS_DEN),
             a1[:, :H] / (a1[:, H:] + EPS_DEN)], axis=1)
        hb = h_ref[...] + upd
        ho_ref[...] = hb
    else:
        (h_ref, wfc_ref, wi_ref, wj_ref, bfc_ref, wv_ref,
         ts_ref, td_ref) = refs
        hb = h_ref[...]
    hl = jnp.dot(hb, wfc_ref[...], preferred_element_type=jnp.float32)
    hl = hl + bfc_ref[0, :]
    aj = jnp.dot(hl, wj_ref[...], preferred_element_type=jnp.float32)
    ai = jnp.dot(hl, wi_ref[...], preferred_element_type=jnp.float32)
    ai = ai + wv_ref[0, :]
    H = EMBED // 2
    ts_ref[0] = jnp.concatenate([aj[:, :H], hl[:, :H]], axis=1)
    ts_ref[1] = jnp.concatenate([aj[:, H:], hl[:, H:]], axis=1)
    td_ref[...] = ai


_BLK = 632
_row_spec = pl.BlockSpec((_BLK, EMBED), lambda r: (r, 0))
_half_spec = pl.BlockSpec((_BLK, EMBED // 2), lambda r: (r, 0))
_w_spec = pl.BlockSpec((EMBED, EMBED), lambda r: (0, 0))
_vec_spec = pl.BlockSpec((1, EMBED), lambda r: (0, 0))
_ts_spec = pl.BlockSpec((2, _BLK, EMBED), lambda r: (0, r, 0))

_dense0_call = pl.pallas_call(
    functools.partial(_dense_body, False),
    grid=(R // _BLK,),
    in_specs=[_row_spec, _w_spec, _w_spec, _w_spec, _vec_spec, _vec_spec],
    out_specs=[_ts_spec, _row_spec],
    out_shape=[jax.ShapeDtypeStruct((2, R, EMBED), jnp.float32),
               jax.ShapeDtypeStruct((R, EMBED), jnp.float32)],
)

_dense1_call = pl.pallas_call(
    functools.partial(_dense_body, True),
    grid=(R // _BLK,),
    in_specs=[_row_spec, _row_spec, _row_spec,
              _w_spec, _w_spec, _w_spec, _vec_spec, _vec_spec],
    out_specs=[_row_spec, _ts_spec, _row_spec],
    out_shape=[jax.ShapeDtypeStruct((R, EMBED), jnp.float32),
               jax.ShapeDtypeStruct((2, R, EMBED), jnp.float32),
               jax.ShapeDtypeStruct((R, EMBED), jnp.float32)],
)


def _final_body(h_ref, a0_ref, a1_ref, ho_ref, mean_ref):
    H = EMBED // 2
    a0 = a0_ref[...]
    a1 = a1_ref[...]
    upd = jnp.concatenate(
        [a0[:, :H] / (a0[:, H:] + EPS_DEN),
         a1[:, :H] / (a1[:, H:] + EPS_DEN)], axis=1)
    hb = h_ref[...] + upd
    ho_ref[...] = hb
    mean_ref[...] = jnp.sum(hb[:N, :], axis=0, keepdims=True) / N


_final_call = pl.pallas_call(
    _final_body,
    out_shape=[jax.ShapeDtypeStruct((R, EMBED), jnp.float32),
               jax.ShapeDtypeStruct((1, EMBED), jnp.float32)],
)


_sc_mesh = plsc.VectorSubcoreMesh(core_axis_name="c", subcore_axis_name="s")
HALF = EMBED // 2


@functools.partial(
    pl.kernel,
    mesh=_sc_mesh,
    out_type=jax.ShapeDtypeStruct((2, R, EMBED), jnp.float32),
    scratch_types=[
        pltpu.VMEM((K,), jnp.int32),
        pltpu.VMEM((K,), jnp.int32),
        pltpu.VMEM((K,), jnp.float32),
        pltpu.VMEM((K, EMBED), jnp.float32),
        pltpu.VMEM((K, EMBED), jnp.float32),
        pltpu.VMEM((K, EMBED), jnp.float32),
        pltpu.VMEM((1, EMBED), jnp.float32),
        pltpu.VMEM_SHARED((R, EMBED), jnp.float32),
        pltpu.SemaphoreType.DMA,
        pltpu.SemaphoreType.DMA,
    ],
)
def _edge_kernel(src_h, dst_h, attr_h, tsrc_h, tdst_h, uv_h,
                 acc_out,
                 srcv, dstv, attrv, srows, drows, nb, uvm, acc, sem1, sem2):
    c = lax.axis_index("c")
    s = lax.axis_index("s")
    rowbase = s * ROWS_PT
    coff = c * R
    coffh = c * HALF

    def _zero_body(i, _):
        zv = jnp.zeros((16,), jnp.float32)
        for cc in range(EMBED // 16):
            nb[i, pl.ds(16 * cc, 16)] = zv
        return 0
    lax.fori_loop(0, K, _zero_body, 0)
    for j in range(ROWS_PT // K):
        pltpu.sync_copy(nb, acc.at[pl.ds(rowbase + j * K, K)])
    _rem = ROWS_PT - (ROWS_PT // K) * K
    if _rem:
        pltpu.sync_copy(nb.at[pl.ds(0, _rem)],
                        acc.at[pl.ds(rowbase + (ROWS_PT // K) * K, _rem)])
    pltpu.sync_copy(uv_h, uvm)
    plsc.subcore_barrier()

    uvecs = [uvm[0, pl.ds(coffh + 16 * cc, 16)] for cc in range(HALF // 16)]
    ebase = s * EPT

    def _chunk(k, _):
        eb = ebase + k * K
        pltpu.sync_copy(src_h.at[pl.ds(eb, K)], srcv)
        pltpu.sync_copy(dst_h.at[pl.ds(eb, K)], dstv)
        pltpu.sync_copy(attr_h.at[pl.ds(eb, K)], attrv)

        def _adj(i, _):
            sl = pl.ds(i * 16, 16)
            srcv[sl] = srcv[sl] + coff
            return 0
        lax.fori_loop(0, K // 16, _adj, 0)

        cp1 = pltpu.async_copy(tsrc_h.at[srcv], srows, sem1)
        cp2 = pltpu.async_copy(tdst_h.at[dstv], drows, sem2)
        cp1.wait()
        cp2.wait()

        def _edge(g, _):
            av16 = attrv[pl.ds(g * 16, 16)]
            for i in range(16):
                e = g * 16 + i
                av = jnp.full((16,), av16[i], jnp.float32)
                for cc in range(HALF // 16):
                    sl = pl.ds(16 * cc, 16)
                    ai = drows[e, pl.ds(coffh + 16 * cc, 16)]
                    t = ai + srows[e, sl] + av * uvecs[cc]
                    t = jnp.maximum(t, 0.2 * t)
                    ex = jnp.exp(t)
                    nb[e, pl.ds(HALF + 16 * cc, 16)] = ex
                    nb[e, sl] = ex * srows[e, pl.ds(HALF + 16 * cc, 16)]
            return 0
        lax.fori_loop(0, K // 16, _edge, 0)

        pltpu.sync_copy(nb, acc.at[dstv], add=True)
        return 0

    lax.fori_loop(0, NCH, _chunk, 0)
    plsc.subcore_barrier()

    pltpu.sync_copy(acc.at[pl.ds(rowbase, ROWS_PT)],
                    acc_out.at[c, pl.ds(rowbase, ROWS_PT)])


def kernel(x, tw, demand, edge_index, edge_attr, Wn, bn_b, g1, b1, We, be_b,
           g2, b2, Wfc, bfc, Wattn, battn):
    f32 = jnp.float32
    z = jnp.concatenate(
        [x, tw, demand, jnp.zeros((N, 3), f32)], axis=1)
    wnt = jnp.concatenate(
        [Wn.T, jnp.zeros((3, EMBED), f32)], axis=0)
    p = jnp.zeros((8, EMBED), f32)
    p = p.at[0, :HE].set(We[:, 0])
    p = p.at[1, :HE].set(g2)
    p = p.at[2, :HE].set(b2)
    p = p.at[3, :].set(bn_b)
    p = p.at[4, :].set(g1)
    p = p.at[5, :].set(b1)
    wea_all = Wattn[:, :, 2 * EMBED:].reshape(LAYERS * EMBED, HE)
    battn2 = jnp.concatenate(
        [battn.reshape(1, LAYERS * EMBED),
         jnp.zeros((7, LAYERS * EMBED), f32)], axis=0)
    attr2d = edge_attr.reshape(E // EMBED, EMBED)

    h, uw = _prep_call(z, attr2d, p, wnt, wea_all, battn2)

    src = edge_index[0]
    dst = edge_index[1]
    pad = E_PAD - E
    src_p = jnp.concatenate([src, jnp.full((pad,), N, jnp.int32)])
    dst_p = jnp.concatenate([dst, jnp.full((pad,), N, jnp.int32)])
    attr_p = jnp.concatenate([edge_attr[:, 0], jnp.zeros((pad,), f32)])

    acc = None
    for l in range(LAYERS):
        wfct = Wfc[l].T
        wit = Wattn[l][:, :EMBED].T
        wjt = Wattn[l][:, EMBED:2 * EMBED].T
        bfcv = bfc[l].reshape(1, EMBED)
        wv = uw[1, l * EMBED:(l + 1) * EMBED].reshape(1, EMBED)
        uv = uw[0, l * EMBED:(l + 1) * EMBED].reshape(1, EMBED)
        if l == 0:
            tsrc, tdst = _dense0_call(h, wfct, wit, wjt, bfcv, wv)
        else:
            h, tsrc, tdst = _dense1_call(
                h, acc[0], acc[1], wfct, wit, wjt, bfcv, wv)
        acc = _edge_kernel(
            src_p, dst_p, attr_p,
            tsrc.reshape(2 * R, EMBED), tdst, uv)

    h_fin, mean = _final_call(h, acc[0], acc[1])
    xr = h_fin[:N].reshape(1, N, EMBED)
    return (xr, mean)


# speedup vs baseline: 3.4767x; 1.2678x over previous
import functools

import jax
import jax.numpy as jnp
from jax import lax
from jax.experimental import pallas as pl
from jax.experimental.pallas import tpu as pltpu
from jax.experimental.pallas import tpu_sc as plsc

N = 10000
E = 320000
EMBED = 128
HE = 16
LAYERS = 3
R = 10112
K = 80
NCH = 250
EPT = NCH * K
E_PAD = 16 * EPT
ROWS_PT = R // 16
EPS_BN = 1e-5
EPS_DEN = 1e-16


def _prep_body(z_ref, attr_ref, p_ref, wnt_ref, wea_ref, battn_ref,
               h_ref, uw_ref):
    z = z_ref[...]
    y = jnp.dot(z, wnt_ref[...], preferred_element_type=jnp.float32)
    y = y + p_ref[3, :]
    m = jnp.mean(y, axis=0)
    v = jnp.mean(y * y, axis=0) - m * m
    h0 = (y - m) / jnp.sqrt(v + EPS_BN) * p_ref[4, :] + p_ref[5, :]
    h_ref[...] = jnp.concatenate(
        [h0, jnp.zeros((R - N, EMBED), jnp.float32)], axis=0)

    a2 = attr_ref[...]
    mean_a = jnp.sum(a2) / E
    var_a = jnp.sum(a2 * a2) / E - mean_a * mean_a
    we = p_ref[0, :HE]
    g2 = p_ref[1, :HE]
    b2 = p_ref[2, :HE]
    s16 = g2 * we / jnp.sqrt(var_a * we * we + EPS_BN)
    t16 = b2 - s16 * mean_a
    u_flat = jnp.dot(wea_ref[...], s16[:, None],
                     preferred_element_type=jnp.float32)[:, 0]
    w_flat = jnp.dot(wea_ref[...], t16[:, None],
                     preferred_element_type=jnp.float32)[:, 0] + battn_ref[0, :]
    uw_ref[...] = jnp.concatenate(
        [u_flat[None, :], w_flat[None, :],
         jnp.zeros((6, LAYERS * EMBED), jnp.float32)], axis=0)


_prep_call = pl.pallas_call(
    _prep_body,
    out_shape=[jax.ShapeDtypeStruct((R, EMBED), jnp.float32),
               jax.ShapeDtypeStruct((8, LAYERS * EMBED), jnp.float32)],
)


def _dense_body(with_update, *refs):
    if with_update:
        (h_ref, a0_ref, a1_ref, wfc_ref, wi_ref, wj_ref,
         bfc_ref, wv_ref, ho_ref, ts_ref, td_ref) = refs
        a0 = a0_ref[...]
        a1 = a1_ref[...]
        H = EMBED // 2
        upd = jnp.concatenate(
            [a0[:, H:] / (a0[:, :H] + EPS_DEN),
             a1[:, H:] / (a1[:, :H] + EPS_DEN)], axis=1)
        hb = h_ref[...] + upd
        ho_ref[...] = hb
    else:
        (h_ref, wfc_ref, wi_ref, wj_ref, bfc_ref, wv_ref,
         ts_ref, td_ref) = refs
        hb = h_ref[...]
    hl = jnp.dot(hb, wfc_ref[...], preferred_element_type=jnp.float32)
    hl = hl + bfc_ref[0, :]
    aj = jnp.dot(hl, wj_ref[...], preferred_element_type=jnp.float32)
    ai = jnp.dot(hl, wi_ref[...], preferred_element_type=jnp.float32)
    ai = ai + wv_ref[0, :]
    H = EMBED // 2
    ts_ref[0] = jnp.concatenate([aj[:, :H], hl[:, :H]], axis=1)
    ts_ref[1] = jnp.concatenate([aj[:, H:], hl[:, H:]], axis=1)
    td_ref[...] = ai


_BLK = 632
_row_spec = pl.BlockSpec((_BLK, EMBED), lambda r: (r, 0))
_half_spec = pl.BlockSpec((_BLK, EMBED // 2), lambda r: (r, 0))
_w_spec = pl.BlockSpec((EMBED, EMBED), lambda r: (0, 0))
_vec_spec = pl.BlockSpec((1, EMBED), lambda r: (0, 0))
_ts_spec = pl.BlockSpec((2, _BLK, EMBED), lambda r: (0, r, 0))

_dense0_call = pl.pallas_call(
    functools.partial(_dense_body, False),
    grid=(R // _BLK,),
    in_specs=[_row_spec, _w_spec, _w_spec, _w_spec, _vec_spec, _vec_spec],
    out_specs=[_ts_spec, _row_spec],
    out_shape=[jax.ShapeDtypeStruct((2, R, EMBED), jnp.float32),
               jax.ShapeDtypeStruct((R, EMBED), jnp.float32)],
)

_dense1_call = pl.pallas_call(
    functools.partial(_dense_body, True),
    grid=(R // _BLK,),
    in_specs=[_row_spec, _row_spec, _row_spec,
              _w_spec, _w_spec, _w_spec, _vec_spec, _vec_spec],
    out_specs=[_row_spec, _ts_spec, _row_spec],
    out_shape=[jax.ShapeDtypeStruct((R, EMBED), jnp.float32),
               jax.ShapeDtypeStruct((2, R, EMBED), jnp.float32),
               jax.ShapeDtypeStruct((R, EMBED), jnp.float32)],
)


def _final_body(h_ref, a0_ref, a1_ref, ho_ref, mean_ref):
    H = EMBED // 2
    a0 = a0_ref[...]
    a1 = a1_ref[...]
    upd = jnp.concatenate(
        [a0[:, H:] / (a0[:, :H] + EPS_DEN),
         a1[:, H:] / (a1[:, :H] + EPS_DEN)], axis=1)
    hb = h_ref[...] + upd
    ho_ref[...] = hb
    mean_ref[...] = jnp.sum(hb[:N, :], axis=0, keepdims=True) / N


_final_call = pl.pallas_call(
    _final_body,
    out_shape=[jax.ShapeDtypeStruct((R, EMBED), jnp.float32),
               jax.ShapeDtypeStruct((1, EMBED), jnp.float32)],
)


_sc_mesh = plsc.VectorSubcoreMesh(core_axis_name="c", subcore_axis_name="s")
HALF = EMBED // 2


@functools.partial(
    pl.kernel,
    mesh=_sc_mesh,
    out_type=jax.ShapeDtypeStruct((2, R, EMBED), jnp.float32),
    scratch_types=[
        pltpu.VMEM((K,), jnp.int32),
        pltpu.VMEM((K,), jnp.int32),
        pltpu.VMEM((K,), jnp.int32),
        pltpu.VMEM((K,), jnp.int32),
        pltpu.VMEM((K,), jnp.float32),
        pltpu.VMEM((K,), jnp.float32),
        pltpu.VMEM((K, EMBED), jnp.float32),
        pltpu.VMEM((K, EMBED), jnp.float32),
        pltpu.VMEM((K, EMBED), jnp.float32),
        pltpu.VMEM((K, EMBED), jnp.float32),
        pltpu.VMEM((1, EMBED), jnp.float32),
        pltpu.VMEM_SHARED((R, EMBED), jnp.float32),
        pltpu.SemaphoreType.DMA,
        pltpu.SemaphoreType.DMA,
        pltpu.SemaphoreType.DMA,
        pltpu.SemaphoreType.DMA,
        pltpu.SemaphoreType.DMA,
        pltpu.SemaphoreType.DMA,
        pltpu.SemaphoreType.DMA,
        pltpu.SemaphoreType.DMA,
        pltpu.SemaphoreType.DMA,
        pltpu.SemaphoreType.DMA,
    ],
)
def _edge_kernel(src_h, dst_h, attr_h, tsrc_h, tdst_h, uv_h,
                 acc_out,
                 sidxA, sidxB, didxA, didxB, attrA, attrB,
                 srowsA, srowsB, drowsA, drowsB, uvm, acc,
                 semSiA, semSiB, semDiA, semDiB, semAtA, semAtB,
                 semSrA, semSrB, semDrA, semDrB):
    c = lax.axis_index("c")
    s = lax.axis_index("s")
    rowbase = s * ROWS_PT
    coff = c * R
    coffh = c * HALF

    def _zero_body(i, _):
        zv = jnp.zeros((16,), jnp.float32)
        for cc in range(EMBED // 16):
            srowsA[i, pl.ds(16 * cc, 16)] = zv
        return 0
    lax.fori_loop(0, K, _zero_body, 0)
    for j in range(ROWS_PT // K):
        pltpu.sync_copy(srowsA, acc.at[pl.ds(rowbase + j * K, K)])
    _rem = ROWS_PT - (ROWS_PT // K) * K
    if _rem:
        pltpu.sync_copy(srowsA.at[pl.ds(0, _rem)],
                        acc.at[pl.ds(rowbase + (ROWS_PT // K) * K, _rem)])
    pltpu.sync_copy(uv_h, uvm)
    plsc.subcore_barrier()

    uvecs = [uvm[0, pl.ds(coffh + 16 * cc, 16)] for cc in range(HALF // 16)]
    ebase = s * EPT

    def _issue_idx(k, sidx, didx, attrv, semSi, semDi, semAt):
        pltpu.async_copy(src_h.at[pl.ds(ebase + k * K, K)], sidx, semSi)
        pltpu.async_copy(dst_h.at[pl.ds(ebase + k * K, K)], didx, semDi)
        pltpu.async_copy(attr_h.at[pl.ds(ebase + k * K, K)], attrv, semAt)

    def _wait_idx(sidx, didx, attrv, semSi, semDi, semAt):
        pltpu.make_async_copy(src_h.at[pl.ds(ebase, K)], sidx, semSi).wait()
        pltpu.make_async_copy(dst_h.at[pl.ds(ebase, K)], didx, semDi).wait()
        pltpu.make_async_copy(attr_h.at[pl.ds(ebase, K)], attrv, semAt).wait()
        for i in range(K // 16):
            sl = pl.ds(i * 16, 16)
            sidx[sl] = sidx[sl] + coff

    def _issue_rows(sidx, didx, srows, drows, semSr, semDr):
        pltpu.async_copy(tsrc_h.at[sidx], srows, semSr)
        pltpu.async_copy(tdst_h.at[didx], drows, semDr)

    def _wait_rows(sidx, didx, srows, drows, semSr, semDr):
        pltpu.make_async_copy(tsrc_h.at[sidx], srows, semSr).wait()
        pltpu.make_async_copy(tdst_h.at[didx], drows, semDr).wait()

    def _compute(didx, srows, drows, attrv):
        def _edge(g, _):
            av16 = attrv[pl.ds(g * 16, 16)]
            for i in range(16):
                e = g * 16 + i
                av = jnp.full((16,), av16[i], jnp.float32)
                for cc in range(HALF // 16):
                    sl = pl.ds(16 * cc, 16)
                    slh = pl.ds(HALF + 16 * cc, 16)
                    ai = drows[e, pl.ds(coffh + 16 * cc, 16)]
                    t = ai + srows[e, sl] + av * uvecs[cc]
                    t = jnp.maximum(t, 0.2 * t)
                    ex = jnp.exp(t)
                    srows[e, sl] = ex
                    srows[e, slh] = ex * srows[e, slh]
            return 0
        lax.fori_loop(0, K // 16, _edge, 0)
        pltpu.sync_copy(srows, acc.at[didx], add=True)

    _wA = (sidxA, didxA, attrA, semSiA, semDiA, semAtA)
    _wB = (sidxB, didxB, attrB, semSiB, semDiB, semAtB)
    _rA = (sidxA, didxA, srowsA, drowsA, semSrA, semDrA)
    _rB = (sidxB, didxB, srowsB, drowsB, semSrB, semDrB)

    _issue_idx(0, *_wA)
    _wait_idx(*_wA)
    _issue_rows(*_rA)
    _issue_idx(1, *_wB)

    def _pair(j, _):
        k = 2 * j
        _wait_idx(*_wB)
        _issue_rows(*_rB)
        _wait_rows(*_rA)
        _compute(didxA, srowsA, drowsA, attrA)
        _issue_idx(k + 2, *_wA)
        _wait_idx(*_wA)
        _issue_rows(*_rA)
        _wait_rows(*_rB)
        _compute(didxB, srowsB, drowsB, attrB)
        _issue_idx(k + 3, *_wB)
        return 0
    lax.fori_loop(0, NCH // 2 - 1, _pair, 0)

    _wait_idx(*_wB)
    _issue_rows(*_rB)
    _wait_rows(*_rA)
    _compute(didxA, srowsA, drowsA, attrA)
    _wait_rows(*_rB)
    _compute(didxB, srowsB, drowsB, attrB)
    plsc.subcore_barrier()

    pltpu.sync_copy(acc.at[pl.ds(rowbase, ROWS_PT)],
                    acc_out.at[c, pl.ds(rowbase, ROWS_PT)])


def kernel(x, tw, demand, edge_index, edge_attr, Wn, bn_b, g1, b1, We, be_b,
           g2, b2, Wfc, bfc, Wattn, battn):
    f32 = jnp.float32
    z = jnp.concatenate(
        [x, tw, demand, jnp.zeros((N, 3), f32)], axis=1)
    wnt = jnp.concatenate(
        [Wn.T, jnp.zeros((3, EMBED), f32)], axis=0)
    p = jnp.zeros((8, EMBED), f32)
    p = p.at[0, :HE].set(We[:, 0])
    p = p.at[1, :HE].set(g2)
    p = p.at[2, :HE].set(b2)
    p = p.at[3, :].set(bn_b)
    p = p.at[4, :].set(g1)
    p = p.at[5, :].set(b1)
    wea_all = Wattn[:, :, 2 * EMBED:].reshape(LAYERS * EMBED, HE)
    battn2 = jnp.concatenate(
        [battn.reshape(1, LAYERS * EMBED),
         jnp.zeros((7, LAYERS * EMBED), f32)], axis=0)
    attr2d = edge_attr.reshape(E // EMBED, EMBED)

    h, uw = _prep_call(z, attr2d, p, wnt, wea_all, battn2)

    src = edge_index[0]
    dst = edge_index[1]
    pad = E_PAD - E
    if pad:
        src_p = jnp.concatenate([src, jnp.full((pad,), N, jnp.int32)])
        dst_p = jnp.concatenate([dst, jnp.full((pad,), N, jnp.int32)])
        attr_p = jnp.concatenate([edge_attr[:, 0], jnp.zeros((pad,), f32)])
    else:
        src_p, dst_p, attr_p = src, dst, edge_attr[:, 0]

    acc = None
    for l in range(LAYERS):
        wfct = Wfc[l].T
        wit = Wattn[l][:, :EMBED].T
        wjt = Wattn[l][:, EMBED:2 * EMBED].T
        bfcv = bfc[l].reshape(1, EMBED)
        wv = uw[1, l * EMBED:(l + 1) * EMBED].reshape(1, EMBED)
        uv = uw[0, l * EMBED:(l + 1) * EMBED].reshape(1, EMBED)
        if l == 0:
            tsrc, tdst = _dense0_call(h, wfct, wit, wjt, bfcv, wv)
        else:
            h, tsrc, tdst = _dense1_call(
                h, acc[0], acc[1], wfct, wit, wjt, bfcv, wv)
        acc = _edge_kernel(
            src_p, dst_p, attr_p,
            tsrc.reshape(2 * R, EMBED), tdst, uv)

    h_fin, mean = _final_call(h, acc[0], acc[1])
    xr = h_fin[:N].reshape(1, N, EMBED)
    return (xr, mean)


# speedup vs baseline: 3.7558x; 1.0803x over previous
import functools

import jax
import jax.numpy as jnp
from jax import lax
from jax.experimental import pallas as pl
from jax.experimental.pallas import tpu as pltpu
from jax.experimental.pallas import tpu_sc as plsc

N = 10000
E = 320000
EMBED = 128
HE = 16
LAYERS = 3
R = 10112
K = 80
NCH = 250
EPT = NCH * K
E_PAD = 16 * EPT
ROWS_PT = R // 16
EPS_BN = 1e-5
EPS_DEN = 1e-16


def _prep_body(z_ref, attr_ref, p_ref, wnt_ref, wea_ref, battn_ref,
               h_ref, uw_ref):
    z = z_ref[...]
    y = jnp.dot(z, wnt_ref[...], preferred_element_type=jnp.float32)
    y = y + p_ref[3, :]
    m = jnp.mean(y, axis=0)
    v = jnp.mean(y * y, axis=0) - m * m
    h0 = (y - m) / jnp.sqrt(v + EPS_BN) * p_ref[4, :] + p_ref[5, :]
    h_ref[...] = jnp.concatenate(
        [h0, jnp.zeros((R - N, EMBED), jnp.float32)], axis=0)

    a2 = attr_ref[...]
    mean_a = jnp.sum(a2) / E
    var_a = jnp.sum(a2 * a2) / E - mean_a * mean_a
    we = p_ref[0, :HE]
    g2 = p_ref[1, :HE]
    b2 = p_ref[2, :HE]
    s16 = g2 * we / jnp.sqrt(var_a * we * we + EPS_BN)
    t16 = b2 - s16 * mean_a
    u_flat = jnp.dot(wea_ref[...], s16[:, None],
                     preferred_element_type=jnp.float32)[:, 0]
    w_flat = jnp.dot(wea_ref[...], t16[:, None],
                     preferred_element_type=jnp.float32)[:, 0] + battn_ref[0, :]
    uw_ref[...] = jnp.concatenate(
        [u_flat[None, :], w_flat[None, :],
         jnp.zeros((6, LAYERS * EMBED), jnp.float32)], axis=0)


_prep_call = pl.pallas_call(
    _prep_body,
    out_shape=[jax.ShapeDtypeStruct((R, EMBED), jnp.float32),
               jax.ShapeDtypeStruct((8, LAYERS * EMBED), jnp.float32)],
)


def _dense_body(with_update, *refs):
    if with_update:
        (h_ref, a0_ref, a1_ref, wfc_ref, wi_ref, wj_ref,
         bfc_ref, wv_ref, ho_ref, ts_ref, td_ref) = refs
        a0 = a0_ref[...]
        a1 = a1_ref[...]
        H = EMBED // 2
        upd = jnp.concatenate(
            [a0[:, H:] / (a0[:, :H] + EPS_DEN),
             a1[:, H:] / (a1[:, :H] + EPS_DEN)], axis=1)
        hb = h_ref[...] + upd
        ho_ref[...] = hb
    else:
        (h_ref, wfc_ref, wi_ref, wj_ref, bfc_ref, wv_ref,
         ts_ref, td_ref) = refs
        hb = h_ref[...]
    hl = jnp.dot(hb, wfc_ref[...], preferred_element_type=jnp.float32)
    hl = hl + bfc_ref[0, :]
    aj = jnp.dot(hl, wj_ref[...], preferred_element_type=jnp.float32)
    ai = jnp.dot(hl, wi_ref[...], preferred_element_type=jnp.float32)
    ai = ai + wv_ref[0, :]
    H = EMBED // 2
    ts_ref[0] = jnp.concatenate([aj[:, :H], hl[:, :H]], axis=1)
    ts_ref[1] = jnp.concatenate([aj[:, H:], hl[:, H:]], axis=1)
    td_ref[...] = ai


_BLK = 632
_row_spec = pl.BlockSpec((_BLK, EMBED), lambda r: (r, 0))
_half_spec = pl.BlockSpec((_BLK, EMBED // 2), lambda r: (r, 0))
_w_spec = pl.BlockSpec((EMBED, EMBED), lambda r: (0, 0))
_vec_spec = pl.BlockSpec((1, EMBED), lambda r: (0, 0))
_ts_spec = pl.BlockSpec((2, _BLK, EMBED), lambda r: (0, r, 0))

_dense0_call = pl.pallas_call(
    functools.partial(_dense_body, False),
    grid=(R // _BLK,),
    in_specs=[_row_spec, _w_spec, _w_spec, _w_spec, _vec_spec, _vec_spec],
    out_specs=[_ts_spec, _row_spec],
    out_shape=[jax.ShapeDtypeStruct((2, R, EMBED), jnp.float32),
               jax.ShapeDtypeStruct((R, EMBED), jnp.float32)],
)

_dense1_call = pl.pallas_call(
    functools.partial(_dense_body, True),
    grid=(R // _BLK,),
    in_specs=[_row_spec, _row_spec, _row_spec,
              _w_spec, _w_spec, _w_spec, _vec_spec, _vec_spec],
    out_specs=[_row_spec, _ts_spec, _row_spec],
    out_shape=[jax.ShapeDtypeStruct((R, EMBED), jnp.float32),
               jax.ShapeDtypeStruct((2, R, EMBED), jnp.float32),
               jax.ShapeDtypeStruct((R, EMBED), jnp.float32)],
)


def _final_body(h_ref, a0_ref, a1_ref, ho_ref, mean_ref):
    H = EMBED // 2
    a0 = a0_ref[...]
    a1 = a1_ref[...]
    upd = jnp.concatenate(
        [a0[:, H:] / (a0[:, :H] + EPS_DEN),
         a1[:, H:] / (a1[:, :H] + EPS_DEN)], axis=1)
    hb = h_ref[...] + upd
    ho_ref[...] = hb
    mean_ref[...] = jnp.sum(hb[:N, :], axis=0, keepdims=True) / N


_final_call = pl.pallas_call(
    _final_body,
    out_shape=[jax.ShapeDtypeStruct((R, EMBED), jnp.float32),
               jax.ShapeDtypeStruct((1, EMBED), jnp.float32)],
)


_sc_mesh = plsc.VectorSubcoreMesh(core_axis_name="c", subcore_axis_name="s")
HALF = EMBED // 2


@functools.partial(
    pl.kernel,
    mesh=_sc_mesh,
    out_type=jax.ShapeDtypeStruct((2, R, EMBED), jnp.float32),
    scratch_types=[
        pltpu.VMEM((K,), jnp.int32),
        pltpu.VMEM((K,), jnp.int32),
        pltpu.VMEM((K,), jnp.int32),
        pltpu.VMEM((K,), jnp.int32),
        pltpu.VMEM((K,), jnp.float32),
        pltpu.VMEM((K,), jnp.float32),
        pltpu.VMEM((K,), jnp.int32),
        pltpu.VMEM((K,), jnp.int32),
        pltpu.VMEM((K,), jnp.float32),
        pltpu.VMEM((K,), jnp.float32),
        pltpu.VMEM((K, EMBED), jnp.float32),
        pltpu.VMEM((K, EMBED), jnp.float32),
        pltpu.VMEM((K, EMBED), jnp.float32),
        pltpu.VMEM((K, EMBED), jnp.float32),
        pltpu.VMEM((1, EMBED), jnp.float32),
        pltpu.VMEM_SHARED((R, EMBED), jnp.float32),
        pltpu.SemaphoreType.DMA,
        pltpu.SemaphoreType.DMA,
        pltpu.SemaphoreType.DMA,
        pltpu.SemaphoreType.DMA,
        pltpu.SemaphoreType.DMA,
        pltpu.SemaphoreType.DMA,
        pltpu.SemaphoreType.DMA,
        pltpu.SemaphoreType.DMA,
        pltpu.SemaphoreType.DMA,
        pltpu.SemaphoreType.DMA,
    ],
)
def _edge_kernel(src_h, dst_h, attr_h, tsrc_h, tdst_h, uv_h,
                 acc_out,
                 sidxA, sidxB, didxA, didxB, attrA, attrB,
                 dsnapA, dsnapB, asnapA, asnapB,
                 srowsA, srowsB, drowsA, drowsB, uvm, acc,
                 semSiA, semSiB, semDiA, semDiB, semAtA, semAtB,
                 semSrA, semSrB, semDrA, semDrB):
    c = lax.axis_index("c")
    s = lax.axis_index("s")
    rowbase = s * ROWS_PT
    coff = c * R
    coffh = c * HALF

    def _zero_body(i, _):
        zv = jnp.zeros((16,), jnp.float32)
        for cc in range(EMBED // 16):
            srowsA[i, pl.ds(16 * cc, 16)] = zv
        return 0
    lax.fori_loop(0, K, _zero_body, 0)
    for j in range(ROWS_PT // K):
        pltpu.sync_copy(srowsA, acc.at[pl.ds(rowbase + j * K, K)])
    _rem = ROWS_PT - (ROWS_PT // K) * K
    if _rem:
        pltpu.sync_copy(srowsA.at[pl.ds(0, _rem)],
                        acc.at[pl.ds(rowbase + (ROWS_PT // K) * K, _rem)])
    pltpu.sync_copy(uv_h, uvm)
    plsc.subcore_barrier()

    uvecs = [uvm[0, pl.ds(coffh + 16 * cc, 16)] for cc in range(HALF // 16)]
    ebase = s * EPT

    def _issue_idx(k, sidx, didx, attrv, semSi, semDi, semAt):
        pltpu.async_copy(src_h.at[pl.ds(ebase + k * K, K)], sidx, semSi)
        pltpu.async_copy(dst_h.at[pl.ds(ebase + k * K, K)], didx, semDi)
        pltpu.async_copy(attr_h.at[pl.ds(ebase + k * K, K)], attrv, semAt)

    def _wait_idx(sidx, didx, attrv, semSi, semDi, semAt):
        pltpu.make_async_copy(src_h.at[pl.ds(ebase, K)], sidx, semSi).wait()
        pltpu.make_async_copy(dst_h.at[pl.ds(ebase, K)], didx, semDi).wait()
        pltpu.make_async_copy(attr_h.at[pl.ds(ebase, K)], attrv, semAt).wait()
        for i in range(K // 16):
            sl = pl.ds(i * 16, 16)
            sidx[sl] = sidx[sl] + coff

    def _issue_rows(sidx, didx, srows, drows, semSr, semDr):
        pltpu.async_copy(tsrc_h.at[sidx], srows, semSr)
        pltpu.async_copy(tdst_h.at[didx], drows, semDr)

    def _wait_rows(sidx, didx, srows, drows, semSr, semDr):
        pltpu.make_async_copy(tsrc_h.at[sidx], srows, semSr).wait()
        pltpu.make_async_copy(tdst_h.at[didx], drows, semDr).wait()

    def _compute(didx, srows, drows, attrv):
        def _edge(g, _):
            av16 = attrv[pl.ds(g * 16, 16)]
            for i in range(16):
                e = g * 16 + i
                av = jnp.full((16,), av16[i], jnp.float32)
                for cc in range(HALF // 16):
                    sl = pl.ds(16 * cc, 16)
                    slh = pl.ds(HALF + 16 * cc, 16)
                    ai = drows[e, pl.ds(coffh + 16 * cc, 16)]
                    t = ai + srows[e, sl] + av * uvecs[cc]
                    t = jnp.maximum(t, 0.2 * t)
                    ex = jnp.exp(t)
                    srows[e, sl] = ex
                    srows[e, slh] = ex * srows[e, slh]
            return 0
        lax.fori_loop(0, K // 16, _edge, 0)
        pltpu.sync_copy(srows, acc.at[didx], add=True)

    def _snap(didx, attrv, dsnap, asnap):
        for i in range(K // 16):
            sl = pl.ds(i * 16, 16)
            dsnap[sl] = didx[sl]
            asnap[sl] = attrv[sl]

    _wA = (sidxA, didxA, attrA, semSiA, semDiA, semAtA)
    _wB = (sidxB, didxB, attrB, semSiB, semDiB, semAtB)
    _rA = (sidxA, didxA, srowsA, drowsA, semSrA, semDrA)
    _rB = (sidxB, didxB, srowsB, drowsB, semSrB, semDrB)

    _issue_idx(0, *_wA)
    _wait_idx(*_wA)
    _issue_rows(*_rA)
    _issue_idx(1, *_wB)

    def _pair(j, _):
        k = 2 * j
        _wait_idx(*_wB)
        _issue_rows(*_rB)
        _wait_rows(*_rA)
        _snap(didxA, attrA, dsnapA, asnapA)
        _issue_idx(k + 2, *_wA)
        _compute(dsnapA, srowsA, drowsA, asnapA)
        _wait_idx(*_wA)
        _issue_rows(*_rA)
        _wait_rows(*_rB)
        _snap(didxB, attrB, dsnapB, asnapB)
        _issue_idx(k + 3, *_wB)
        _compute(dsnapB, srowsB, drowsB, asnapB)
        return 0
    lax.fori_loop(0, NCH // 2 - 1, _pair, 0)

    _wait_idx(*_wB)
    _issue_rows(*_rB)
    _wait_rows(*_rA)
    _compute(didxA, srowsA, drowsA, attrA)
    _wait_rows(*_rB)
    _compute(didxB, srowsB, drowsB, attrB)
    plsc.subcore_barrier()

    pltpu.sync_copy(acc.at[pl.ds(rowbase, ROWS_PT)],
                    acc_out.at[c, pl.ds(rowbase, ROWS_PT)])


def kernel(x, tw, demand, edge_index, edge_attr, Wn, bn_b, g1, b1, We, be_b,
           g2, b2, Wfc, bfc, Wattn, battn):
    f32 = jnp.float32
    z = jnp.concatenate(
        [x, tw, demand, jnp.zeros((N, 3), f32)], axis=1)
    wnt = jnp.concatenate(
        [Wn.T, jnp.zeros((3, EMBED), f32)], axis=0)
    p = jnp.zeros((8, EMBED), f32)
    p = p.at[0, :HE].set(We[:, 0])
    p = p.at[1, :HE].set(g2)
    p = p.at[2, :HE].set(b2)
    p = p.at[3, :].set(bn_b)
    p = p.at[4, :].set(g1)
    p = p.at[5, :].set(b1)
    wea_all = Wattn[:, :, 2 * EMBED:].reshape(LAYERS * EMBED, HE)
    battn2 = jnp.concatenate(
        [battn.reshape(1, LAYERS * EMBED),
         jnp.zeros((7, LAYERS * EMBED), f32)], axis=0)
    attr2d = edge_attr.reshape(E // EMBED, EMBED)

    h, uw = _prep_call(z, attr2d, p, wnt, wea_all, battn2)

    src = edge_index[0]
    dst = edge_index[1]
    pad = E_PAD - E
    if pad:
        src_p = jnp.concatenate([src, jnp.full((pad,), N, jnp.int32)])
        dst_p = jnp.concatenate([dst, jnp.full((pad,), N, jnp.int32)])
        attr_p = jnp.concatenate([edge_attr[:, 0], jnp.zeros((pad,), f32)])
    else:
        src_p, dst_p, attr_p = src, dst, edge_attr[:, 0]

    acc = None
    for l in range(LAYERS):
        wfct = Wfc[l].T
        wit = Wattn[l][:, :EMBED].T
        wjt = Wattn[l][:, EMBED:2 * EMBED].T
        bfcv = bfc[l].reshape(1, EMBED)
        wv = uw[1, l * EMBED:(l + 1) * EMBED].reshape(1, EMBED)
        uv = uw[0, l * EMBED:(l + 1) * EMBED].reshape(1, EMBED)
        if l == 0:
            tsrc, tdst = _dense0_call(h, wfct, wit, wjt, bfcv, wv)
        else:
            h, tsrc, tdst = _dense1_call(
                h, acc[0], acc[1], wfct, wit, wjt, bfcv, wv)
        acc = _edge_kernel(
            src_p, dst_p, attr_p,
            tsrc.reshape(2 * R, EMBED), tdst, uv)

    h_fin, mean = _final_call(h, acc[0], acc[1])
    xr = h_fin[:N].reshape(1, N, EMBED)
    return (xr, mean)
